# baseline (speedup 1.0000x reference)
import functools

import jax
import jax.numpy as jnp
from jax import lax
from jax.experimental import pallas as pl
from jax.experimental.pallas import tpu as pltpu
from jax.experimental.pallas import tpu_sc as plsc

N_NODES = 10000
NNZ = 320000
D = 128
NG = 8

NC, NS = 2, 16
NW = NC * NS
CH = 128
NCHUNK = 80
PER_W = NCHUNK * CH
NNZ_PAD = NW * PER_W
R = 10240
SENT = N_NODES
TZ = R // NS
BLK = 512
GRID = R // BLK
BN_INV = (1.0 + 1e-5) ** -0.5


GB = 8
PAIRCH = 2 * NCHUNK
NCHT = NS * PAIRCH
NCH0 = NCHUNK
NGRP = NCHUNK // GB
NBUF = 2


def _seg_body(table, gidx, sidx, zf,
              out_s,
              gi, si, rows0, rows1,
              g0, g1, s0, s1, acc):
    c = lax.axis_index("c")
    s = lax.axis_index("s")
    base = s * TZ
    cbase = s * PAIRCH + c * NCH0
    pltpu.sync_copy(zf.at[pl.ds(base, TZ)], acc.at[pl.ds(base, TZ)])
    plsc.subcore_barrier()
    rows = (rows0, rows1)
    gsems = (g0, g1)
    ssems = (s0, s1)

    def grp(g, carry):
        pltpu.sync_copy(gidx.at[pl.ds(cbase + g * GB, GB)], gi)
        pltpu.sync_copy(sidx.at[pl.ds(cbase + g * GB, GB)], si)
        gh = [None] * NBUF
        sh = [None] * NBUF
        gh[0] = pltpu.async_copy(table.at[gi.at[0]], rows[0], gsems[0])
        for b in range(GB):
            gh[b % NBUF].wait()
            sh[b % NBUF] = pltpu.async_copy(rows[b % NBUF],
                                            acc.at[si.at[b]], ssems[b % NBUF],
                                            add=True)
            if b + 1 < GB:
                if b >= 1:
                    sh[(b - 1) % NBUF].wait()
                gh[(b + 1) % NBUF] = pltpu.async_copy(
                    table.at[gi.at[b + 1]], rows[(b + 1) % NBUF],
                    gsems[(b + 1) % NBUF])
        for b in range(GB - NBUF, GB):
            sh[b % NBUF].wait()
        return carry

    lax.fori_loop(0, NGRP, grp, 0)
    plsc.subcore_barrier()
    pltpu.sync_copy(acc.at[pl.ds(base, TZ)], out_s.at[pl.ds(c * R + base, TZ)])


def _sc_mesh():
    return plsc.VectorSubcoreMesh(core_axis_name="c", subcore_axis_name="s",
                                  num_cores=NC, num_subcores=NS)


def _seg_call(table, gidx, sidx, zf):
    f = pl.kernel(
        _seg_body,
        out_type=jax.ShapeDtypeStruct((NC * R, D), jnp.float32),
        mesh=_sc_mesh(),
        scratch_types=[
            pltpu.VMEM((GB, CH), jnp.int32),
            pltpu.VMEM((GB, CH), jnp.int32),
            pltpu.VMEM((CH, D), jnp.float32),
            pltpu.VMEM((CH, D), jnp.float32),
            pltpu.SemaphoreType.DMA,
            pltpu.SemaphoreType.DMA,
            pltpu.SemaphoreType.DMA,
            pltpu.SemaphoreType.DMA,
            pltpu.VMEM_SHARED((R, D), jnp.float32),
        ],
    )
    return f(table, gidx, sidx, zf)


HB = 2048
NHB = NNZ_PAD // HB
RA = R // D


def _hist_body(nir, nic, eir, eic, cn_ref, ce_ref):
    pid = pl.program_id(0)

    @pl.when(pid == 0)
    def _():
        cn_ref[...] = jnp.zeros((RA, D), jnp.float32)
        ce_ref[...] = jnp.zeros((RA, D), jnp.float32)

    for r3, c3, out in ((nir, nic, cn_ref), (eir, eic, ce_ref)):
        row = r3[0]
        col = c3[0]
        a_oh = (lax.broadcasted_iota(jnp.int32, (RA, 1), 0) == row // D)
        b_oh = (col % D == lax.broadcasted_iota(jnp.int32, (1, D), 1))
        out[...] += lax.dot_general(
            a_oh.astype(jnp.bfloat16), b_oh.astype(jnp.bfloat16),
            (((1,), (0,)), ((), ())), preferred_element_type=jnp.float32)


def _hist_call(nir, nic, eir, eic):
    return pl.pallas_call(
        _hist_body,
        grid=(NHB,),
        in_specs=[
            pl.BlockSpec((1, 1, HB), lambda i: (i, 0, 0)),
            pl.BlockSpec((1, HB, 1), lambda i: (i, 0, 0)),
            pl.BlockSpec((1, 1, HB), lambda i: (i, 0, 0)),
            pl.BlockSpec((1, HB, 1), lambda i: (i, 0, 0)),
        ],
        out_specs=[
            pl.BlockSpec((RA, D), lambda i: (0, 0)),
            pl.BlockSpec((RA, D), lambda i: (0, 0)),
        ],
        out_shape=[
            jax.ShapeDtypeStruct((RA, D), jnp.float32),
            jax.ShapeDtypeStruct((RA, D), jnp.float32),
        ],
    )(nir, nic, eir, eic)


def _tca_body(x_ref, w1_ref, b1_ref, wc1_ref, o_ref):
    pid = pl.program_id(0)
    h = jnp.maximum(x_ref[...] @ w1_ref[...] + b1_ref[...], 0.0)
    t = h @ wc1_ref[...]
    row = pid * BLK + lax.broadcasted_iota(jnp.int32, (BLK, 1), 0)
    o_ref[...] = jnp.where(row < N_NODES, t, 0.0)


def _tca_call(xp, W1, b1, Wc1):
    return pl.pallas_call(
        _tca_body,
        grid=(GRID,),
        in_specs=[
            pl.BlockSpec((BLK, D), lambda i: (i, 0)),
            pl.BlockSpec((D, D), lambda i: (0, 0)),
            pl.BlockSpec((1, D), lambda i: (0, 0)),
            pl.BlockSpec((D, D), lambda i: (0, 0)),
        ],
        out_specs=pl.BlockSpec((BLK, D), lambda i: (i, 0)),
        out_shape=jax.ShapeDtypeStruct((R, D), jnp.float32),
    )(xp, W1, b1, Wc1)


def _comb_edge_body(s_ref, ce_ref, o_ref):
    ssum = s_ref[0] + s_ref[1]
    cnt = ce_ref[...]
    binv = jnp.where(cnt > 0, 1.0 / cnt, 0.0)
    o_ref[...] = ssum * binv


def _comb_edge_call(S, ce):
    return pl.pallas_call(
        _comb_edge_body,
        grid=(GRID,),
        in_specs=[
            pl.BlockSpec((NC, BLK, D), lambda i: (0, i, 0)),
            pl.BlockSpec((BLK, 1), lambda i: (i, 0)),
        ],
        out_specs=pl.BlockSpec((BLK, D), lambda i: (i, 0)),
        out_shape=jax.ShapeDtypeStruct((R, D), jnp.float32),
    )(S, ce)


def _cd_body(s_ref, cn_ref, bc1_ref, wc2_ref, batch_ref, t2_ref, p1_ref):
    pid = pl.program_id(0)
    ssum = s_ref[0] + s_ref[1]
    cnt = cn_ref[...]
    dinv = jnp.where(cnt > 0, 1.0 / cnt, 0.0)
    h = jnp.maximum(ssum * dinv + bc1_ref[...], 0.0)
    row = pid * BLK + lax.broadcasted_iota(jnp.int32, (BLK, 1), 0)
    h = jnp.where(row < N_NODES, h, 0.0)
    t2_ref[...] = h @ wc2_ref[...]
    onehot = (batch_ref[...] == lax.broadcasted_iota(jnp.int32, (1, NG), 1))
    onehot = onehot.astype(jnp.float32)
    pp = lax.dot_general(onehot, h, (((0,), (0,)), ((), ())))

    @pl.when(pid == 0)
    def _():
        p1_ref[...] = jnp.zeros((NG, D), jnp.float32)

    p1_ref[...] += pp


def _cd_call(S, cn, bc1, Wc2, batchp):
    return pl.pallas_call(
        _cd_body,
        grid=(GRID,),
        in_specs=[
            pl.BlockSpec((NC, BLK, D), lambda i: (0, i, 0)),
            pl.BlockSpec((BLK, 1), lambda i: (i, 0)),
            pl.BlockSpec((1, D), lambda i: (0, 0)),
            pl.BlockSpec((D, D), lambda i: (0, 0)),
            pl.BlockSpec((BLK, 1), lambda i: (i, 0)),
        ],
        out_specs=[
            pl.BlockSpec((BLK, D), lambda i: (i, 0)),
            pl.BlockSpec((NG, D), lambda i: (0, 0)),
        ],
        out_shape=[
            jax.ShapeDtypeStruct((R, D), jnp.float32),
            jax.ShapeDtypeStruct((NG, D), jnp.float32),
        ],
    )(S, cn, bc1, Wc2, batchp)


def _e_body(s_ref, cn_ref, bc2_ref, batch_ref, p1_ref,
            wl_ref, bl_ref, wm1_ref, bm1_ref, g1_ref, be1_ref,
            wm2_ref, bm2_ref, g2_ref, be2_ref, wout_ref,
            o_ref, p2_acc, c_acc):
    pid = pl.program_id(0)
    ssum = s_ref[0] + s_ref[1]
    cnt = cn_ref[...]
    dinv = jnp.where(cnt > 0, 1.0 / cnt, 0.0)
    h = jnp.maximum(ssum * dinv + bc2_ref[...], 0.0)
    onehot = (batch_ref[...] == lax.broadcasted_iota(jnp.int32, (1, NG), 1))
    onehot = onehot.astype(jnp.float32)
    pp = lax.dot_general(onehot, h, (((0,), (0,)), ((), ())))
    cc = jnp.sum(onehot, axis=0)[:, None]

    @pl.when(pid == 0)
    def _():
        p2_acc[...] = jnp.zeros((NG, D), jnp.float32)
        c_acc[...] = jnp.zeros((NG, 1), jnp.float32)

    p2_acc[...] += pp
    c_acc[...] += cc

    @pl.when(pid == GRID - 1)
    def _():
        cdiv = jnp.maximum(c_acc[...], 1.0)
        p1 = p1_ref[...] / cdiv
        p2 = p2_acc[...] / cdiv
        gcat = jnp.concatenate([p1, p2], axis=1)
        out = gcat @ wl_ref[...] + bl_ref[...]
        z = out @ wm1_ref[...] + bm1_ref[...]
        z = jnp.maximum(z * BN_INV * g1_ref[...] + be1_ref[...], 0.0)
        z = z @ wm2_ref[...] + bm2_ref[...]
        z = jnp.maximum(z * BN_INV * g2_ref[...] + be2_ref[...], 0.0)
        o_ref[...] = z @ wout_ref[...]


def _e_call(S, cn, bc2, batchp, p1, Wl, bl, Wm1, bm1, g1, be1,
            Wm2, bm2, g2, be2, Wout):
    const = lambda i: (0, 0)
    return pl.pallas_call(
        _e_body,
        grid=(GRID,),
        in_specs=[
            pl.BlockSpec((NC, BLK, D), lambda i: (0, i, 0)),
            pl.BlockSpec((BLK, 1), lambda i: (i, 0)),
            pl.BlockSpec((1, D), const),
            pl.BlockSpec((BLK, 1), lambda i: (i, 0)),
            pl.BlockSpec((NG, D), const),
            pl.BlockSpec((2 * D, D), const),
            pl.BlockSpec((1, D), const),
            pl.BlockSpec((D, 64), const),
            pl.BlockSpec((1, 64), const),
            pl.BlockSpec((1, 64), const),
            pl.BlockSpec((1, 64), const),
            pl.BlockSpec((64, 32), const),
            pl.BlockSpec((1, 32), const),
            pl.BlockSpec((1, 32), const),
            pl.BlockSpec((1, 32), const),
            pl.BlockSpec((32, 4), const),
        ],
        out_specs=pl.BlockSpec((NG, 4), const),
        out_shape=jax.ShapeDtypeStruct((NG, 4), jnp.float32),
        scratch_shapes=[
            pltpu.VMEM((NG, D), jnp.float32),
            pltpu.VMEM((NG, 1), jnp.float32),
        ],
    )(S, cn, bc2, batchp, p1, Wl, bl, Wm1, bm1, g1, be1,
      Wm2, bm2, g2, be2, Wout)


def kernel(x, edge_index, batch, W1, b1, Wc1, bc1, Wc2, bc2, Wl, bl,
           Wm1, bm1, g1, be1, Wm2, bm2, g2, be2, Wout):
    f32 = jnp.float32
    xp = jnp.pad(x, ((0, R - N_NODES), (0, 0)))
    pad = jnp.full((NNZ_PAD - NNZ,), SENT, jnp.int32)
    ni = jnp.concatenate([edge_index[0], pad])
    ei = jnp.concatenate([edge_index[1], pad])
    niP = ni.reshape(NCHT, CH)
    eiP = ei.reshape(NCHT, CH)
    batchp = jnp.pad(batch, (0, R - N_NODES),
                     constant_values=NG).reshape(R, 1)
    zf = jnp.zeros((R, D), f32)

    b1r = b1.reshape(1, D)
    bc1r = bc1.reshape(1, D)
    bc2r = bc2.reshape(1, D)
    blr = bl.reshape(1, D)
    bm1r = bm1.reshape(1, 64)
    g1r = g1.reshape(1, 64)
    be1r = be1.reshape(1, 64)
    bm2r = bm2.reshape(1, 32)
    g2r = g2.reshape(1, 32)
    be2r = be2.reshape(1, 32)

    CN80, CE80 = _hist_call(ni.reshape(NHB, 1, HB), ni.reshape(NHB, HB, 1),
                            ei.reshape(NHB, 1, HB), ei.reshape(NHB, HB, 1))
    cn = CN80.reshape(R, 1)
    ce = CE80.reshape(R, 1)

    t1 = _tca_call(xp, W1, b1r, Wc1)
    S1 = _seg_call(t1, niP, eiP, zf)
    ef1 = _comb_edge_call(S1.reshape(NC, R, D), ce)
    S2 = _seg_call(ef1, eiP, niP, zf)
    t2, p1 = _cd_call(S2.reshape(NC, R, D), cn, bc1r, Wc2, batchp)

    S3 = _seg_call(t2, niP, eiP, zf)
    ef2 = _comb_edge_call(S3.reshape(NC, R, D), ce)
    S4 = _seg_call(ef2, eiP, niP, zf)

    return _e_call(S4.reshape(NC, R, D), cn, bc2r, batchp, p1,
                   Wl, blr, Wm1, bm1r, g1r, be1r, Wm2, bm2r, g2r, be2r, Wout)


# speedup vs baseline: 1.2702x; 1.2702x over previous
import functools

import jax
import jax.numpy as jnp
from jax import lax
from jax.experimental import pallas as pl
from jax.experimental.pallas import tpu as pltpu
from jax.experimental.pallas import tpu_sc as plsc

N_NODES = 10000
NNZ = 320000
D = 128
NG = 8

NC, NS = 2, 16
NW = NC * NS
CH = 128
NCHUNK = 80
PER_W = NCHUNK * CH
NNZ_PAD = NW * PER_W
R = 10240
SENT = N_NODES
TZ = R // NS
BLK = 512
GRID = R // BLK
BN_INV = (1.0 + 1e-5) ** -0.5


GB = 8
NGRP = NCHUNK // GB


def _seg_body(table, gidx, sidx, zf,
              out_s,
              gi, si, rows0, rows1, sem0, sem1, acc):
    c = lax.axis_index("c")
    s = lax.axis_index("s")
    wid = s * NC + c
    base = s * TZ
    pltpu.sync_copy(zf.at[pl.ds(base, TZ)], acc.at[pl.ds(base, TZ)])
    plsc.subcore_barrier()
    rows = (rows0, rows1)
    sems = (sem0, sem1)

    def grp(g, carry):
        pltpu.sync_copy(gidx.at[wid, pl.ds(g * GB, GB)], gi)
        pltpu.sync_copy(sidx.at[wid, pl.ds(g * GB, GB)], si)
        handles = [None, None]
        handles[0] = pltpu.async_copy(table.at[gi.at[0]], rows[0], sems[0])
        for b in range(GB):
            if b + 1 < GB:
                handles[(b + 1) % 2] = pltpu.async_copy(
                    table.at[gi.at[b + 1]], rows[(b + 1) % 2], sems[(b + 1) % 2])
            handles[b % 2].wait()
            pltpu.sync_copy(rows[b % 2], acc.at[si.at[b]], add=True)
        return carry

    lax.fori_loop(0, NGRP, grp, 0)
    plsc.subcore_barrier()
    pltpu.sync_copy(acc.at[pl.ds(base, TZ)], out_s.at[pl.ds(c * R + base, TZ)])


def _sc_mesh():
    return plsc.VectorSubcoreMesh(core_axis_name="c", subcore_axis_name="s",
                                  num_cores=NC, num_subcores=NS)


def _seg_call(table, gidx, sidx, zf):
    f = pl.kernel(
        _seg_body,
        out_type=jax.ShapeDtypeStruct((NC * R, D), jnp.float32),
        mesh=_sc_mesh(),
        scratch_types=[
            pltpu.VMEM((GB, CH), jnp.int32),
            pltpu.VMEM((GB, CH), jnp.int32),
            pltpu.VMEM((CH, D), jnp.float32),
            pltpu.VMEM((CH, D), jnp.float32),
            pltpu.SemaphoreType.DMA,
            pltpu.SemaphoreType.DMA,
            pltpu.VMEM_SHARED((R, D), jnp.float32),
        ],
    )
    return f(table, gidx, sidx, zf)


HB = 2048
NHB = NNZ_PAD // HB
RA = R // D


def _hist_body(nir, nic, eir, eic, cn_ref, ce_ref):
    pid = pl.program_id(0)

    @pl.when(pid == 0)
    def _():
        cn_ref[...] = jnp.zeros((RA, D), jnp.float32)
        ce_ref[...] = jnp.zeros((RA, D), jnp.float32)

    for r3, c3, out in ((nir, nic, cn_ref), (eir, eic, ce_ref)):
        row = r3[0]
        col = c3[0]
        a_oh = (lax.broadcasted_iota(jnp.int32, (RA, 1), 0) == row // D)
        b_oh = (col % D == lax.broadcasted_iota(jnp.int32, (1, D), 1))
        out[...] += lax.dot_general(
            a_oh.astype(jnp.bfloat16), b_oh.astype(jnp.bfloat16),
            (((1,), (0,)), ((), ())), preferred_element_type=jnp.float32)


def _hist_call(nir, nic, eir, eic):
    return pl.pallas_call(
        _hist_body,
        grid=(NHB,),
        in_specs=[
            pl.BlockSpec((1, 1, HB), lambda i: (i, 0, 0)),
            pl.BlockSpec((1, HB, 1), lambda i: (i, 0, 0)),
            pl.BlockSpec((1, 1, HB), lambda i: (i, 0, 0)),
            pl.BlockSpec((1, HB, 1), lambda i: (i, 0, 0)),
        ],
        out_specs=[
            pl.BlockSpec((RA, D), lambda i: (0, 0)),
            pl.BlockSpec((RA, D), lambda i: (0, 0)),
        ],
        out_shape=[
            jax.ShapeDtypeStruct((RA, D), jnp.float32),
            jax.ShapeDtypeStruct((RA, D), jnp.float32),
        ],
    )(nir, nic, eir, eic)


def _tca_body(x_ref, w1_ref, b1_ref, wc1_ref, o_ref):
    pid = pl.program_id(0)
    h = jnp.maximum(x_ref[...] @ w1_ref[...] + b1_ref[...], 0.0)
    t = h @ wc1_ref[...]
    row = pid * BLK + lax.broadcasted_iota(jnp.int32, (BLK, 1), 0)
    o_ref[...] = jnp.where(row < N_NODES, t, 0.0)


def _tca_call(xp, W1, b1, Wc1):
    return pl.pallas_call(
        _tca_body,
        grid=(GRID,),
        in_specs=[
            pl.BlockSpec((BLK, D), lambda i: (i, 0)),
            pl.BlockSpec((D, D), lambda i: (0, 0)),
            pl.BlockSpec((1, D), lambda i: (0, 0)),
            pl.BlockSpec((D, D), lambda i: (0, 0)),
        ],
        out_specs=pl.BlockSpec((BLK, D), lambda i: (i, 0)),
        out_shape=jax.ShapeDtypeStruct((R, D), jnp.float32),
    )(xp, W1, b1, Wc1)


def _comb_edge_body(s_ref, ce_ref, o_ref):
    ssum = s_ref[0] + s_ref[1]
    cnt = ce_ref[...]
    binv = jnp.where(cnt > 0, 1.0 / cnt, 0.0)
    o_ref[...] = ssum * binv


def _comb_edge_call(S, ce):
    return pl.pallas_call(
        _comb_edge_body,
        grid=(GRID,),
        in_specs=[
            pl.BlockSpec((NC, BLK, D), lambda i: (0, i, 0)),
            pl.BlockSpec((BLK, 1), lambda i: (i, 0)),
        ],
        out_specs=pl.BlockSpec((BLK, D), lambda i: (i, 0)),
        out_shape=jax.ShapeDtypeStruct((R, D), jnp.float32),
    )(S, ce)


def _cd_body(s_ref, cn_ref, bc1_ref, wc2_ref, batch_ref, t2_ref, p1_ref):
    pid = pl.program_id(0)
    ssum = s_ref[0] + s_ref[1]
    cnt = cn_ref[...]
    dinv = jnp.where(cnt > 0, 1.0 / cnt, 0.0)
    h = jnp.maximum(ssum * dinv + bc1_ref[...], 0.0)
    row = pid * BLK + lax.broadcasted_iota(jnp.int32, (BLK, 1), 0)
    h = jnp.where(row < N_NODES, h, 0.0)
    t2_ref[...] = h @ wc2_ref[...]
    onehot = (batch_ref[...] == lax.broadcasted_iota(jnp.int32, (1, NG), 1))
    onehot = onehot.astype(jnp.float32)
    pp = lax.dot_general(onehot, h, (((0,), (0,)), ((), ())))

    @pl.when(pid == 0)
    def _():
        p1_ref[...] = jnp.zeros((NG, D), jnp.float32)

    p1_ref[...] += pp


def _cd_call(S, cn, bc1, Wc2, batchp):
    return pl.pallas_call(
        _cd_body,
        grid=(GRID,),
        in_specs=[
            pl.BlockSpec((NC, BLK, D), lambda i: (0, i, 0)),
            pl.BlockSpec((BLK, 1), lambda i: (i, 0)),
            pl.BlockSpec((1, D), lambda i: (0, 0)),
            pl.BlockSpec((D, D), lambda i: (0, 0)),
            pl.BlockSpec((BLK, 1), lambda i: (i, 0)),
        ],
        out_specs=[
            pl.BlockSpec((BLK, D), lambda i: (i, 0)),
            pl.BlockSpec((NG, D), lambda i: (0, 0)),
        ],
        out_shape=[
            jax.ShapeDtypeStruct((R, D), jnp.float32),
            jax.ShapeDtypeStruct((NG, D), jnp.float32),
        ],
    )(S, cn, bc1, Wc2, batchp)


def _e_body(s_ref, cn_ref, bc2_ref, batch_ref, p1_ref,
            wl_ref, bl_ref, wm1_ref, bm1_ref, g1_ref, be1_ref,
            wm2_ref, bm2_ref, g2_ref, be2_ref, wout_ref,
            o_ref, p2_acc, c_acc):
    pid = pl.program_id(0)
    ssum = s_ref[0] + s_ref[1]
    cnt = cn_ref[...]
    dinv = jnp.where(cnt > 0, 1.0 / cnt, 0.0)
    h = jnp.maximum(ssum * dinv + bc2_ref[...], 0.0)
    onehot = (batch_ref[...] == lax.broadcasted_iota(jnp.int32, (1, NG), 1))
    onehot = onehot.astype(jnp.float32)
    pp = lax.dot_general(onehot, h, (((0,), (0,)), ((), ())))
    cc = jnp.sum(onehot, axis=0)[:, None]

    @pl.when(pid == 0)
    def _():
        p2_acc[...] = jnp.zeros((NG, D), jnp.float32)
        c_acc[...] = jnp.zeros((NG, 1), jnp.float32)

    p2_acc[...] += pp
    c_acc[...] += cc

    @pl.when(pid == GRID - 1)
    def _():
        cdiv = jnp.maximum(c_acc[...], 1.0)
        p1 = p1_ref[...] / cdiv
        p2 = p2_acc[...] / cdiv
        gcat = jnp.concatenate([p1, p2], axis=1)
        out = gcat @ wl_ref[...] + bl_ref[...]
        z = out @ wm1_ref[...] + bm1_ref[...]
        z = jnp.maximum(z * BN_INV * g1_ref[...] + be1_ref[...], 0.0)
        z = z @ wm2_ref[...] + bm2_ref[...]
        z = jnp.maximum(z * BN_INV * g2_ref[...] + be2_ref[...], 0.0)
        o_ref[...] = z @ wout_ref[...]


def _e_call(S, cn, bc2, batchp, p1, Wl, bl, Wm1, bm1, g1, be1,
            Wm2, bm2, g2, be2, Wout):
    const = lambda i: (0, 0)
    return pl.pallas_call(
        _e_body,
        grid=(GRID,),
        in_specs=[
            pl.BlockSpec((NC, BLK, D), lambda i: (0, i, 0)),
            pl.BlockSpec((BLK, 1), lambda i: (i, 0)),
            pl.BlockSpec((1, D), const),
            pl.BlockSpec((BLK, 1), lambda i: (i, 0)),
            pl.BlockSpec((NG, D), const),
            pl.BlockSpec((2 * D, D), const),
            pl.BlockSpec((1, D), const),
            pl.BlockSpec((D, 64), const),
            pl.BlockSpec((1, 64), const),
            pl.BlockSpec((1, 64), const),
            pl.BlockSpec((1, 64), const),
            pl.BlockSpec((64, 32), const),
            pl.BlockSpec((1, 32), const),
            pl.BlockSpec((1, 32), const),
            pl.BlockSpec((1, 32), const),
            pl.BlockSpec((32, 4), const),
        ],
        out_specs=pl.BlockSpec((NG, 4), const),
        out_shape=jax.ShapeDtypeStruct((NG, 4), jnp.float32),
        scratch_shapes=[
            pltpu.VMEM((NG, D), jnp.float32),
            pltpu.VMEM((NG, 1), jnp.float32),
        ],
    )(S, cn, bc2, batchp, p1, Wl, bl, Wm1, bm1, g1, be1,
      Wm2, bm2, g2, be2, Wout)


def kernel(x, edge_index, batch, W1, b1, Wc1, bc1, Wc2, bc2, Wl, bl,
           Wm1, bm1, g1, be1, Wm2, bm2, g2, be2, Wout):
    f32 = jnp.float32
    xp = jnp.pad(x, ((0, R - N_NODES), (0, 0)))
    pad = jnp.full((NNZ_PAD - NNZ,), SENT, jnp.int32)
    ni = jnp.concatenate([edge_index[0], pad])
    ei = jnp.concatenate([edge_index[1], pad])
    niP = ni.reshape(NW, NCHUNK, CH)
    eiP = ei.reshape(NW, NCHUNK, CH)
    batchp = jnp.pad(batch, (0, R - N_NODES),
                     constant_values=NG).reshape(R, 1)
    zf = jnp.zeros((R, D), f32)

    b1r = b1.reshape(1, D)
    bc1r = bc1.reshape(1, D)
    bc2r = bc2.reshape(1, D)
    blr = bl.reshape(1, D)
    bm1r = bm1.reshape(1, 64)
    g1r = g1.reshape(1, 64)
    be1r = be1.reshape(1, 64)
    bm2r = bm2.reshape(1, 32)
    g2r = g2.reshape(1, 32)
    be2r = be2.reshape(1, 32)

    CN80, CE80 = _hist_call(ni.reshape(NHB, 1, HB), ni.reshape(NHB, HB, 1),
                            ei.reshape(NHB, 1, HB), ei.reshape(NHB, HB, 1))
    cn = CN80.reshape(R, 1)
    ce = CE80.reshape(R, 1)

    t1 = _tca_call(xp, W1, b1r, Wc1)
    S1 = _seg_call(t1, niP, eiP, zf)
    ef1 = _comb_edge_call(S1.reshape(NC, R, D), ce)
    S2 = _seg_call(ef1, eiP, niP, zf)
    t2, p1 = _cd_call(S2.reshape(NC, R, D), cn, bc1r, Wc2, batchp)

    S3 = _seg_call(t2, niP, eiP, zf)
    ef2 = _comb_edge_call(S3.reshape(NC, R, D), ce)
    S4 = _seg_call(ef2, eiP, niP, zf)

    return _e_call(S4.reshape(NC, R, D), cn, bc2r, batchp, p1,
                   Wl, blr, Wm1, bm1r, g1r, be1r, Wm2, bm2r, g2r, be2r, Wout)
